# baseline (speedup 1.0000x reference)
import functools

import jax
import jax.numpy as jnp
import numpy as np
from jax.experimental import pallas as pl

_NP = 32


def _leaky(x):
    return jnp.maximum(x, 0.01 * x)


def _conv_chain(h, ws):
    for t in range(4):
        h = jnp.dot(h, ws[2 * t], preferred_element_type=jnp.float32) + ws[2 * t + 1]
        h = _leaky(h)
    return jnp.dot(h, ws[8], preferred_element_type=jnp.float32) + ws[9]


def _fwd_kernel(n_real, bb, pp, x_ref, pit_ref, pj_ref, gf_ref, *refs):
    prefs, sig_ref, logit_ref = refs[:-2], refs[-2], refs[-1]
    p = [r[...] for r in prefs]
    wc0, wc1, wcl = p[0:10], p[10:20], p[20:30]
    fc0w, fc0b, fc1w, fc1b, fclw, fclb = p[30:36]

    x = x_ref[...]
    pit = pit_ref[...]
    pj = pj_ref[...]
    gf = gf_ref[...]

    for wc, fcw, fcb in ((wc0, fc0w, fc0b), (wc1, fc1w, fc1b)):
        f = x.shape[-1]
        x2d = x.reshape(bb * _NP, f)
        d = jnp.abs(jnp.concatenate(
            [jnp.dot(gf, x2d[c * 4 * _NP:(c + 1) * 4 * _NP, :],
                     preferred_element_type=jnp.float32)
             for c in range(bb // 4)], axis=0))
        lp = _conv_chain(d, wc).reshape(bb, pp)
        ep = jnp.exp(lp)
        scaled = ep[:, None, :] * pit[None, :, :]
        u = jnp.dot(scaled.reshape(bb * _NP, pp), pj,
                    preferred_element_type=jnp.float32).reshape(bb, _NP, _NP)
        emat = u + u.transpose(0, 2, 1)
        s = jnp.sum(emat, axis=2, keepdims=True)
        a = emat / jnp.maximum(s, 1e-30)
        agg = jax.lax.dot_general(a, x, (((2,), (1,)), ((0,), (0,))),
                                  preferred_element_type=jnp.float32)
        h = jnp.concatenate([x, agg], axis=-1).reshape(bb * _NP, 2 * f)
        h = jnp.dot(h, fcw, preferred_element_type=jnp.float32) + fcb
        x = jnp.concatenate([x, _leaky(h).reshape(bb, _NP, -1)], axis=-1)

    f = x.shape[-1]
    d0 = jnp.abs(x[:, 0:1, :] - x)
    lw0 = _conv_chain(d0.reshape(bb * _NP, f), wcl).reshape(bb, _NP)
    jcol = jax.lax.broadcasted_iota(jnp.int32, (bb, _NP), 1)
    lw0 = lw0 - jnp.where(jcol == 0, 1e8, 0.0) - jnp.where(jcol >= n_real, 1e9, 0.0)
    m0 = jnp.max(lw0, axis=1, keepdims=True)
    e0 = jnp.exp(lw0 - m0)
    a0 = e0 / jnp.sum(e0, axis=1, keepdims=True)
    agg0 = jnp.sum(a0[:, :, None] * x, axis=1)
    h0 = jnp.concatenate([x[:, 0, :], agg0], axis=-1)
    logits = jnp.dot(h0, fclw, preferred_element_type=jnp.float32) + fclb
    sig_ref[...] = 1.0 / (1.0 + jnp.exp(-logits))
    logit_ref[...] = logits


def _fold_conv(p, i):
    s = p['bn%d_g' % i] / jnp.sqrt(1.0 + 1e-5)
    w = p['c%d_w' % i] * s[:, None]
    b = p['c%d_b' % i] * s + p['bn%d_b' % i]
    return w.T, b[None, :]


def _flatten_wcomp(p):
    out = []
    for i in range(1, 5):
        wt, b = _fold_conv(p, i)
        out += [wt, b]
    out += [p['cl_w'].T, p['cl_b'][None, :]]
    return out


def _flatten_gconv(p, bn):
    w, b = p['fc_w'], p['fc_b']
    if bn:
        s = p['bn_g'] / jnp.sqrt(1.0 + 1e-5)
        w = w * s[:, None]
        b = b * s + p['bn_b']
    return [w.T, b[None, :]]


def kernel(z, zi_s, labels_yi, params):
    bsz, emb = z.shape
    s, _, nw = labels_yi.shape
    n = s + 1

    zero_pad = jnp.zeros_like(labels_yi[0])
    labels_all = jnp.concatenate([zero_pad[None], labels_yi], axis=0)
    zi_all = jnp.concatenate([z[None], zi_s], axis=0)
    nodes = jnp.concatenate([zi_all, labels_all], axis=-1)
    x = jnp.transpose(nodes, (1, 0, 2))
    x = jnp.pad(x, ((0, 0), (0, _NP - n), (0, 0)))

    flat = (_flatten_wcomp(params['w0']) + _flatten_wcomp(params['w1'])
            + _flatten_wcomp(params['wl'])
            + _flatten_gconv(params['l0'], True)
            + _flatten_gconv(params['l1'], True)
            + _flatten_gconv(params['ll'], False))

    iu, ju = np.triu_indices(n, 1)
    pp = -(-len(iu) // 8) * 8
    iu = np.pad(iu, (0, pp - len(iu)))
    ju = np.pad(ju, (0, pp - len(ju)))
    valid = (np.arange(pp) < len(np.triu_indices(n, 1)[0]))[:, None]
    pi_np = np.eye(_NP, dtype=np.float32)[iu] * valid
    pj_np = np.eye(_NP, dtype=np.float32)[ju] * valid
    pit = jnp.asarray(pi_np.T.copy())
    pj = jnp.asarray(pj_np)

    bb = 32
    gf = jnp.asarray(np.kron(np.eye(4, dtype=np.float32), pi_np - pj_np))
    f0 = x.shape[-1]
    grid = (bsz // bb,)
    in_specs = [pl.BlockSpec((bb, _NP, f0), lambda i: (i, 0, 0)),
                pl.BlockSpec((_NP, pp), lambda i: (0, 0)),
                pl.BlockSpec((pp, _NP), lambda i: (0, 0)),
                pl.BlockSpec((4 * pp, 4 * _NP), lambda i: (0, 0))]
    in_specs += [pl.BlockSpec(a.shape, lambda i, _nd=a.ndim: (0,) * _nd)
                 for a in flat]
    out_specs = [pl.BlockSpec((bb, nw), lambda i: (i, 0))] * 2
    out_shape = [jax.ShapeDtypeStruct((bsz, nw), jnp.float32)] * 2

    sig, logits = pl.pallas_call(
        functools.partial(_fwd_kernel, n, bb, pp),
        grid=grid,
        in_specs=in_specs,
        out_specs=out_specs,
        out_shape=out_shape,
    )(x, pit, pj, gf, *flat)
    return sig, logits


# speedup vs baseline: 3.1849x; 3.1849x over previous
import functools

import jax
import jax.numpy as jnp
import numpy as np
from jax.experimental import pallas as pl

_NP = 32


def _leaky(x):
    return jnp.maximum(x, 0.01 * x)


def _conv_chain(h, ws):
    for t in range(4):
        h = jnp.dot(h, ws[2 * t], preferred_element_type=jnp.float32) + ws[2 * t + 1]
        h = _leaky(h)
    return jnp.dot(h, ws[8], preferred_element_type=jnp.float32) + ws[9]


def _fwd_kernel(n_real, bb, pp, x_ref, pi_ref, pj_ref, gf_ref, *refs):
    prefs, sig_ref, logit_ref = refs[:-2], refs[-2], refs[-1]
    p = [r[...] for r in prefs]
    wc0, wc1, wcl = p[0:10], p[10:20], p[20:30]
    fc0w, fc0b, fc1w, fc1b, fclw, fclb = p[30:36]

    x = x_ref[...]
    pi = pi_ref[...]
    pj = pj_ref[...]
    gf = gf_ref[...]
    pjb = jnp.broadcast_to(pj[None], (bb, pp, _NP))

    for wc, fcw, fcb in ((wc0, fc0w, fc0b), (wc1, fc1w, fc1b)):
        f = x.shape[-1]
        x2d = x.reshape(bb * _NP, f)
        d = jnp.abs(jnp.concatenate(
            [jnp.dot(gf, x2d[c * 8 * _NP:(c + 1) * 8 * _NP, :],
                     preferred_element_type=jnp.float32)
             for c in range(bb // 8)], axis=0))
        lp = _conv_chain(d, wc).reshape(bb, pp)
        ep = jnp.exp(lp)
        scaled = pi[None] * ep[:, :, None]
        u = jax.lax.dot_general(scaled, pjb, (((1,), (1,)), ((0,), (0,))),
                                preferred_element_type=jnp.float32)
        emat = u + u.transpose(0, 2, 1)
        s = jnp.sum(emat, axis=2, keepdims=True)
        a = emat / jnp.maximum(s, 1e-30)
        agg = jax.lax.dot_general(a, x, (((2,), (1,)), ((0,), (0,))),
                                  preferred_element_type=jnp.float32)
        h = jnp.concatenate([x, agg], axis=-1).reshape(bb * _NP, 2 * f)
        h = jnp.dot(h, fcw, preferred_element_type=jnp.float32) + fcb
        x = jnp.concatenate([x, _leaky(h).reshape(bb, _NP, -1)], axis=-1)

    f = x.shape[-1]
    d0 = jnp.abs(x[:, 0:1, :] - x)
    lw0 = _conv_chain(d0.reshape(bb * _NP, f), wcl).reshape(bb, _NP)
    jcol = jax.lax.broadcasted_iota(jnp.int32, (bb, _NP), 1)
    lw0 = lw0 - jnp.where(jcol == 0, 1e8, 0.0) - jnp.where(jcol >= n_real, 1e9, 0.0)
    m0 = jnp.max(lw0, axis=1, keepdims=True)
    e0 = jnp.exp(lw0 - m0)
    a0 = e0 / jnp.sum(e0, axis=1, keepdims=True)
    agg0 = jnp.sum(a0[:, :, None] * x, axis=1)
    h0 = jnp.concatenate([x[:, 0, :], agg0], axis=-1)
    logits = jnp.dot(h0, fclw, preferred_element_type=jnp.float32) + fclb
    sig_ref[...] = 1.0 / (1.0 + jnp.exp(-logits))
    logit_ref[...] = logits


def _fold_conv(p, i):
    s = p['bn%d_g' % i] / jnp.sqrt(1.0 + 1e-5)
    w = p['c%d_w' % i] * s[:, None]
    b = p['c%d_b' % i] * s + p['bn%d_b' % i]
    return w.T, b[None, :]


def _flatten_wcomp(p):
    out = []
    for i in range(1, 5):
        wt, b = _fold_conv(p, i)
        out += [wt, b]
    out += [p['cl_w'].T, p['cl_b'][None, :]]
    return out


def _flatten_gconv(p, bn):
    w, b = p['fc_w'], p['fc_b']
    if bn:
        s = p['bn_g'] / jnp.sqrt(1.0 + 1e-5)
        w = w * s[:, None]
        b = b * s + p['bn_b']
    return [w.T, b[None, :]]


def kernel(z, zi_s, labels_yi, params):
    bsz, emb = z.shape
    s, _, nw = labels_yi.shape
    n = s + 1

    zero_pad = jnp.zeros_like(labels_yi[0])
    labels_all = jnp.concatenate([zero_pad[None], labels_yi], axis=0)
    zi_all = jnp.concatenate([z[None], zi_s], axis=0)
    nodes = jnp.concatenate([zi_all, labels_all], axis=-1)
    x = jnp.transpose(nodes, (1, 0, 2))
    x = jnp.pad(x, ((0, 0), (0, _NP - n), (0, 0)))

    flat = (_flatten_wcomp(params['w0']) + _flatten_wcomp(params['w1'])
            + _flatten_wcomp(params['wl'])
            + _flatten_gconv(params['l0'], True)
            + _flatten_gconv(params['l1'], True)
            + _flatten_gconv(params['ll'], False))

    iu, ju = np.triu_indices(n, 1)
    pp = -(-len(iu) // 8) * 8
    iu = np.pad(iu, (0, pp - len(iu)))
    ju = np.pad(ju, (0, pp - len(ju)))
    valid = (np.arange(pp) < len(np.triu_indices(n, 1)[0]))[:, None]
    pi_np = np.eye(_NP, dtype=np.float32)[iu] * valid
    pj_np = np.eye(_NP, dtype=np.float32)[ju] * valid
    pi = jnp.asarray(pi_np)
    pj = jnp.asarray(pj_np)

    bb = 32
    gf = jnp.asarray(np.kron(np.eye(8, dtype=np.float32), pi_np - pj_np))
    f0 = x.shape[-1]
    grid = (bsz // bb,)
    in_specs = [pl.BlockSpec((bb, _NP, f0), lambda i: (i, 0, 0)),
                pl.BlockSpec((pp, _NP), lambda i: (0, 0)),
                pl.BlockSpec((pp, _NP), lambda i: (0, 0)),
                pl.BlockSpec((8 * pp, 8 * _NP), lambda i: (0, 0))]
    in_specs += [pl.BlockSpec(a.shape, lambda i, _nd=a.ndim: (0,) * _nd)
                 for a in flat]
    out_specs = [pl.BlockSpec((bb, nw), lambda i: (i, 0))] * 2
    out_shape = [jax.ShapeDtypeStruct((bsz, nw), jnp.float32)] * 2

    sig, logits = pl.pallas_call(
        functools.partial(_fwd_kernel, n, bb, pp),
        grid=grid,
        in_specs=in_specs,
        out_specs=out_specs,
        out_shape=out_shape,
    )(x, pi, pj, gf, *flat)
    return sig, logits
